# Initial kernel scaffold; baseline (speedup 1.0000x reference)
import jax
import jax.numpy as jnp
from jax.experimental import pallas as pl


def kernel(x, edge_index, W1, att_src1, att_dst1, b1, W2, att_src2, att_dst2, b2, W_fc, b_fc):
    raise NotImplementedError("write your pallas kernel here")


# speedup vs baseline: 40.9691x; 40.9691x over previous
import functools

import jax
import jax.numpy as jnp
from jax import lax
from jax.experimental import pallas as pl
from jax.experimental.pallas import tpu as pltpu
from jax.experimental.pallas import tpu_sc as plsc

N = 10000
E = 320000
NPAD = 10240
NS = 16
NW = 32
EPT = 10240
EPAD = NW * EPT
RPT = NPAD // NS

BE = 64
SB = 16
NCH_E = EPT // BE
NSB = NCH_E // SB

BS = 128
NCH_S = EPT // BS

f32 = jnp.float32
i32 = jnp.int32

_sc_mesh = plsc.VectorSubcoreMesh(core_axis_name="c", subcore_axis_name="s")


@functools.partial(
    pl.kernel,
    out_type=(jax.ShapeDtypeStruct((2 * NPAD, 128), f32),
              jax.ShapeDtypeStruct((2, NPAD), f32)),
    mesh=_sc_mesh,
    compiler_params=pltpu.CompilerParams(needs_layout_passes=False),
    scratch_types=[
        pltpu.VMEM((2 * NPAD,), f32),
        pltpu.VMEM((SB, BE), i32),
        pltpu.VMEM((SB, BE), i32),
        pltpu.VMEM((SB, BE), f32),
        pltpu.VMEM((BE, 128), f32),
        pltpu.VMEM((BE, 128), f32),
        pltpu.VMEM((RPT,), f32),
        pltpu.VMEM_SHARED((NPAD, 128), f32),
        pltpu.VMEM_SHARED((NPAD,), f32),
        pltpu.SemaphoreType.DMA,
        pltpu.SemaphoreType.DMA,
    ],
)
def _sc_edge(h_hbm, a_hbm, src_hbm, dst_hbm, nump_hbm, denp_hbm,
             a_tab, sblk, dblk, wblk, rows0, rows1, zbuf,
             num_sp, den_sp, sem0, sem1):
    c = lax.axis_index("c")
    s = lax.axis_index("s")
    g = c * NS + s

    pltpu.sync_copy(a_hbm, a_tab)

    zv = jnp.zeros((16,), f32)

    @pl.loop(0, BE)
    def _zero_rows(r):
        for k in range(8):
            rows0[r, pl.ds(k * 16, 16)] = zv

    @pl.loop(0, RPT // 16)
    def _zero_z(r):
        zbuf[pl.ds(r * 16, 16)] = zv

    for q in range(RPT // BE):
        pltpu.sync_copy(rows0, num_sp.at[pl.ds(s * RPT + q * BE, BE)])
    pltpu.sync_copy(zbuf, den_sp.at[pl.ds(s * RPT, RPT)])

    plsc.subcore_barrier()

    iota = lax.iota(i32, 16)

    def _weights(b, j):
        base = g * EPT + (b * SB + j) * BE
        for q in range(BE // 16):
            off = q * 16
            s16 = sblk[j, pl.ds(off, 16)]
            d16 = dblk[j, pl.ds(off, 16)]
            va = plsc.load_gather(a_tab, [s16 * 2])
            vb = plsc.load_gather(a_tab, [d16 * 2 + 1])
            e = va + vb
            e = jnp.where(e > 0, e, 0.2 * e)
            w = jnp.exp(e)
            w = jnp.where(base + off + iota < E, w, 0.0)
            wblk[j, pl.ds(off, 16)] = w

    def _start(j, rows, sem):
        pltpu.async_copy(h_hbm.at[sblk.at[j]], rows, sem)

    def _wait(j, rows, sem):
        pltpu.make_async_copy(h_hbm.at[sblk.at[j]], rows, sem).wait()

    def _scale(j, rows):
        @pl.loop(0, BE // 16)
        def _sc(q):
            w16 = wblk[j, pl.ds(q * 16, 16)]
            for t in range(16):
                wv = jnp.full((16,), w16[t])
                r = q * 16 + t
                for k in range(8):
                    sl = pl.ds(k * 16, 16)
                    rows[r, sl] = rows[r, sl] * wv

    def _scatter(j, rows):
        pltpu.sync_copy(rows, num_sp.at[dblk.at[j]], add=True)
        pltpu.sync_copy(wblk.at[j], den_sp.at[dblk.at[j]], add=True)

    @pl.loop(0, NSB)
    def _super(b):
        pltpu.sync_copy(src_hbm.at[g, pl.ds(b * SB, SB)], sblk)
        pltpu.sync_copy(dst_hbm.at[g, pl.ds(b * SB, SB)], dblk)
        _start(0, rows0, sem0)

        @pl.loop(0, SB, step=2)
        def _pair(j):
            _start(j + 1, rows1, sem1)
            _weights(b, j)
            _wait(j, rows0, sem0)
            _scale(j, rows0)
            _scatter(j, rows0)

            @pl.when(j + 2 < SB)
            def _():
                _start(j + 2, rows0, sem0)

            _weights(b, j + 1)
            _wait(j + 1, rows1, sem1)
            _scale(j + 1, rows1)
            _scatter(j + 1, rows1)

    plsc.subcore_barrier()
    pltpu.sync_copy(num_sp.at[pl.ds(s * RPT, RPT)],
                    nump_hbm.at[pl.ds(c * NPAD + s * RPT, RPT)])
    pltpu.sync_copy(den_sp.at[pl.ds(s * RPT, RPT)],
                    denp_hbm.at[c, pl.ds(s * RPT, RPT)])


@functools.partial(
    pl.kernel,
    out_type=(jax.ShapeDtypeStruct((NW, NCH_S, BS), f32),
              jax.ShapeDtypeStruct((NW, NCH_S, BS), f32)),
    mesh=_sc_mesh,
    compiler_params=pltpu.CompilerParams(needs_layout_passes=False),
    scratch_types=[
        pltpu.VMEM((NPAD * 4,), f32),
        pltpu.VMEM((NCH_S, BS), i32),
        pltpu.VMEM((NCH_S, BS), i32),
        pltpu.VMEM((NCH_S, BS), f32),
        pltpu.VMEM((NCH_S, BS), f32),
    ],
)
def _sc_score(uv_hbm, src_hbm, dst_hbm, s0_hbm, s1_hbm,
              uv_tab, src_idx, dst_idx, s0buf, s1buf):
    c = lax.axis_index("c")
    s = lax.axis_index("s")
    g = c * NS + s
    pltpu.sync_copy(uv_hbm, uv_tab)
    pltpu.sync_copy(src_hbm.at[g], src_idx)
    pltpu.sync_copy(dst_hbm.at[g], dst_idx)

    @pl.loop(0, NCH_S)
    def _lp(j):
        for k in range(BS // 16):
            off = k * 16
            s16 = src_idx[j, pl.ds(off, 16)] * 4
            d16 = dst_idx[j, pl.ds(off, 16)] * 4
            u0 = plsc.load_gather(uv_tab, [s16])
            u1 = plsc.load_gather(uv_tab, [s16 + 1])
            v0 = plsc.load_gather(uv_tab, [d16 + 2])
            v1 = plsc.load_gather(uv_tab, [d16 + 3])
            s0buf[j, pl.ds(off, 16)] = u0 + v0
            s1buf[j, pl.ds(off, 16)] = u1 + v1

    pltpu.sync_copy(s0buf, s0_hbm.at[g])
    pltpu.sync_copy(s1buf, s1_hbm.at[g])


BM = 1024
NBLK = NPAD // BM


def _tc_dense1(x, W, attc):
    def body(x_ref, w_ref, att_ref, h_ref, a_ref):
        h = jnp.dot(x_ref[...], w_ref[...], preferred_element_type=f32)
        h_ref[...] = h
        a_ref[...] = jnp.dot(h, att_ref[...], preferred_element_type=f32)

    return pl.pallas_call(
        body,
        grid=(NBLK,),
        in_specs=[pl.BlockSpec((BM, 128), lambda i: (i, 0)),
                  pl.BlockSpec((128, 128), lambda i: (0, 0)),
                  pl.BlockSpec((128, 2), lambda i: (0, 0))],
        out_specs=[pl.BlockSpec((BM, 128), lambda i: (i, 0)),
                   pl.BlockSpec((BM, 2), lambda i: (i, 0))],
        out_shape=(jax.ShapeDtypeStruct((NPAD, 128), f32),
                   jax.ShapeDtypeStruct((NPAD, 2), f32)),
    )(x, W, attc)


def _combine(n0_ref, n1_ref, d0_ref, d1_ref, b_ref):
    num = n0_ref[...] + n1_ref[...]
    den = d0_ref[...] + d1_ref[...]
    z = num / (den + 1e-16) + b_ref[...]
    return jnp.where(z > 0, z, jnp.exp(z) - 1.0)


def _tc_combine_dense(nump, den0, den1, bias, W, attc):
    def body(n0_ref, n1_ref, d0_ref, d1_ref, b_ref, w_ref, att_ref,
             h_ref, a_ref):
        hin = _combine(n0_ref, n1_ref, d0_ref, d1_ref, b_ref)
        h = jnp.dot(hin, w_ref[...], preferred_element_type=f32)
        h_ref[...] = h
        a_ref[...] = jnp.dot(h, att_ref[...], preferred_element_type=f32)

    return pl.pallas_call(
        body,
        grid=(NBLK,),
        in_specs=[pl.BlockSpec((BM, 128), lambda i: (i, 0)),
                  pl.BlockSpec((BM, 128), lambda i: (i + NBLK, 0)),
                  pl.BlockSpec((BM, 1), lambda i: (i, 0)),
                  pl.BlockSpec((BM, 1), lambda i: (i, 0)),
                  pl.BlockSpec((1, 128), lambda i: (0, 0)),
                  pl.BlockSpec((128, 128), lambda i: (0, 0)),
                  pl.BlockSpec((128, 2), lambda i: (0, 0))],
        out_specs=[pl.BlockSpec((BM, 128), lambda i: (i, 0)),
                   pl.BlockSpec((BM, 2), lambda i: (i, 0))],
        out_shape=(jax.ShapeDtypeStruct((NPAD, 128), f32),
                   jax.ShapeDtypeStruct((NPAD, 2), f32)),
    )(nump, nump, den0, den1, bias, W, attc)


def _tc_combine_proj(nump, den0, den1, bias, Wcat, bcat):
    def body(n0_ref, n1_ref, d0_ref, d1_ref, b_ref, w_ref, bc_ref, o_ref):
        hf = _combine(n0_ref, n1_ref, d0_ref, d1_ref, b_ref)
        o_ref[...] = (jnp.dot(hf, w_ref[...], preferred_element_type=f32)
                      + bc_ref[...])

    return pl.pallas_call(
        body,
        grid=(NBLK,),
        in_specs=[pl.BlockSpec((BM, 128), lambda i: (i, 0)),
                  pl.BlockSpec((BM, 128), lambda i: (i + NBLK, 0)),
                  pl.BlockSpec((BM, 1), lambda i: (i, 0)),
                  pl.BlockSpec((BM, 1), lambda i: (i, 0)),
                  pl.BlockSpec((1, 128), lambda i: (0, 0)),
                  pl.BlockSpec((128, 4), lambda i: (0, 0)),
                  pl.BlockSpec((1, 4), lambda i: (0, 0))],
        out_specs=pl.BlockSpec((BM, 4), lambda i: (i, 0)),
        out_shape=jax.ShapeDtypeStruct((NPAD, 4), f32),
    )(nump, nump, den0, den1, bias, Wcat, bcat)


def _tc_logsm(s0, s1):
    rows = EPAD // BS

    def body(s0_ref, s1_ref, o0_ref, o1_ref):
        a = s0_ref[...]
        b = s1_ref[...]
        m = jnp.maximum(a, b)
        mn = jnp.minimum(a, b)
        l = m + jnp.log(1.0 + jnp.exp(mn - m))
        o0_ref[...] = a - l
        o1_ref[...] = b - l

    bm = 512
    return pl.pallas_call(
        body,
        grid=(rows // bm,),
        in_specs=[pl.BlockSpec((bm, BS), lambda i: (i, 0)),
                  pl.BlockSpec((bm, BS), lambda i: (i, 0))],
        out_specs=[pl.BlockSpec((bm, BS), lambda i: (i, 0)),
                   pl.BlockSpec((bm, BS), lambda i: (i, 0))],
        out_shape=(jax.ShapeDtypeStruct((rows, BS), f32),
                   jax.ShapeDtypeStruct((rows, BS), f32)),
    )(s0, s1)


def kernel(x, edge_index, W1, att_src1, att_dst1, b1,
           W2, att_src2, att_dst2, b2, W_fc, b_fc):
    x_pad = jnp.pad(x, ((0, NPAD - N), (0, 0)))
    npad_e = EPAD - E
    pad_dst = (N + (jnp.arange(npad_e, dtype=i32) % (NPAD - N))).astype(i32)
    src_all = jnp.concatenate([edge_index[0], pad_dst])
    dst_all = jnp.concatenate([edge_index[1], pad_dst])
    src_e = src_all.reshape(NW, NCH_E, BE)
    dst_e = dst_all.reshape(NW, NCH_E, BE)
    src_r = src_all.reshape(NW, NCH_S, BS)
    dst_r = dst_all.reshape(NW, NCH_S, BS)

    attc1 = jnp.stack([att_src1.reshape(-1), att_dst1.reshape(-1)], axis=1)
    attc2 = jnp.stack([att_src2.reshape(-1), att_dst2.reshape(-1)], axis=1)
    b1r = b1.reshape(1, 128)
    b2r = b2.reshape(1, 128)
    Wcat = jnp.concatenate([W_fc[:128], W_fc[128:]], axis=1)
    bcat = jnp.concatenate([b_fc, jnp.zeros((2,), f32)]).reshape(1, 4)

    h1, a1 = _tc_dense1(x_pad, W1, attc1)
    nump1, denp1 = _sc_edge(h1, a1.reshape(-1), src_e, dst_e)
    d10 = denp1[0].reshape(NPAD, 1)
    d11 = denp1[1].reshape(NPAD, 1)
    h2, a2 = _tc_combine_dense(nump1, d10, d11, b1r, W2, attc2)
    nump2, denp2 = _sc_edge(h2, a2.reshape(-1), src_e, dst_e)
    d20 = denp2[0].reshape(NPAD, 1)
    d21 = denp2[1].reshape(NPAD, 1)
    uv = _tc_combine_proj(nump2, d20, d21, b2r, Wcat, bcat)
    s0, s1 = _sc_score(uv.reshape(-1), src_r, dst_r)
    o0, o1 = _tc_logsm(s0.reshape(EPAD // BS, BS), s1.reshape(EPAD // BS, BS))
    return jnp.stack([o0.reshape(EPAD)[:E], o1.reshape(EPAD)[:E]], axis=1)


# speedup vs baseline: 41.9222x; 1.0233x over previous
import functools

import jax
import jax.numpy as jnp
from jax import lax
from jax.experimental import pallas as pl
from jax.experimental.pallas import tpu as pltpu
from jax.experimental.pallas import tpu_sc as plsc

N = 10000
E = 320000
NPAD = 10240
NS = 16
NW = 32
EPT = 10240
EPAD = NW * EPT
RPT = NPAD // NS

BE = 64
SB = 8
NCH_E = EPT // BE
NSB = NCH_E // SB

BS = 128
NCH_S = EPT // BS

f32 = jnp.float32
i32 = jnp.int32

_sc_mesh = plsc.VectorSubcoreMesh(core_axis_name="c", subcore_axis_name="s")


@functools.partial(
    pl.kernel,
    out_type=(jax.ShapeDtypeStruct((2 * NPAD, 128), f32),
              jax.ShapeDtypeStruct((2, NPAD), f32)),
    mesh=_sc_mesh,
    compiler_params=pltpu.CompilerParams(needs_layout_passes=False),
    scratch_types=[
        pltpu.VMEM((2 * NPAD,), f32),
        pltpu.VMEM((SB, BE), i32),
        pltpu.VMEM((SB, BE), i32),
        pltpu.VMEM((SB, BE), f32),
        pltpu.VMEM((BE, 128), f32),
        pltpu.VMEM((BE, 128), f32),
        pltpu.VMEM((BE, 128), f32),
        pltpu.VMEM_SHARED((NPAD, 128), f32),
        pltpu.VMEM_SHARED((NPAD,), f32),
        pltpu.SemaphoreType.DMA,
        pltpu.SemaphoreType.DMA,
        pltpu.SemaphoreType.DMA,
        pltpu.SemaphoreType.DMA,
        pltpu.SemaphoreType.DMA,
        pltpu.SemaphoreType.DMA,
        pltpu.SemaphoreType.DMA,
    ],
)
def _sc_edge(h_hbm, a_hbm, src_hbm, dst_hbm, nump_hbm, denp_hbm,
             a_tab, sblk, dblk, wblk, rows0, rows1, rows2,
             num_sp, den_sp, gs0, gs1, gs2, ss0, ss1, ss2, dsem):
    c = lax.axis_index("c")
    s = lax.axis_index("s")
    g = c * NS + s

    pltpu.sync_copy(a_hbm, a_tab)

    zv = jnp.zeros((16,), f32)

    @pl.loop(0, BE)
    def _zero_rows(r):
        for k in range(8):
            rows0[r, pl.ds(k * 16, 16)] = zv

    for q in range(RPT // BE):
        pltpu.sync_copy(rows0, num_sp.at[pl.ds(s * RPT + q * BE, BE)])
    for r in range(RPT // 128):
        pltpu.sync_copy(rows0.at[r], den_sp.at[pl.ds(s * RPT + r * 128, 128)])

    plsc.subcore_barrier()

    iota = lax.iota(i32, 16)

    def _weights(b, j):
        base = g * EPT + (b * SB + j) * BE
        for q in range(BE // 16):
            off = q * 16
            s16 = sblk[j, pl.ds(off, 16)]
            d16 = dblk[j, pl.ds(off, 16)]
            va = plsc.load_gather(a_tab, [s16 * 2])
            vb = plsc.load_gather(a_tab, [d16 * 2 + 1])
            e = va + vb
            e = jnp.where(e > 0, e, 0.2 * e)
            w = jnp.exp(e)
            w = jnp.where(base + off + iota < E, w, 0.0)
            wblk[j, pl.ds(off, 16)] = w

    ring = ((rows0, gs0, ss0), (rows1, gs1, ss1), (rows2, gs2, ss2))

    def _start(j, rows, sem):
        pltpu.async_copy(h_hbm.at[sblk.at[j]], rows, sem)

    def _wait_gather(j, rows, sem):
        pltpu.make_async_copy(h_hbm.at[sblk.at[j]], rows, sem).wait()

    def _wait_scatter(rows, sem):
        pltpu.make_async_copy(rows, num_sp.at[dblk.at[0]], sem).wait()

    def _drain_den():
        for _ in range(SB):
            pltpu.make_async_copy(wblk.at[0], den_sp.at[dblk.at[0]],
                                  dsem).wait()

    def _scale(j, rows):
        @pl.loop(0, BE // 16)
        def _sc(q):
            w16 = wblk[j, pl.ds(q * 16, 16)]
            for t in range(16):
                wv = jnp.full((16,), w16[t])
                r = q * 16 + t
                for k in range(8):
                    sl = pl.ds(k * 16, 16)
                    rows[r, sl] = rows[r, sl] * wv

    @pl.loop(0, NSB)
    def _super(b):
        @pl.when(b > 0)
        def _drain_prev():
            for rows, _, ssem in ring:
                _wait_scatter(rows, ssem)
            _drain_den()

        pltpu.sync_copy(src_hbm.at[g, pl.ds(b * SB, SB)], sblk)
        pltpu.sync_copy(dst_hbm.at[g, pl.ds(b * SB, SB)], dblk)
        _start(0, rows0, gs0)
        _start(1, rows1, gs1)

        for j in range(SB):
            rows, gsem, ssem = ring[j % 3]
            _weights(b, j)
            _wait_gather(j, rows, gsem)
            _scale(j, rows)
            pltpu.async_copy(rows, num_sp.at[dblk.at[j]], ssem, add=True)
            pltpu.async_copy(wblk.at[j], den_sp.at[dblk.at[j]], dsem,
                             add=True)
            if j + 2 < SB:
                nrows, ngsem, nssem = ring[(j + 2) % 3]
                if j + 2 >= 3:
                    _wait_scatter(nrows, nssem)
                _start(j + 2, nrows, ngsem)

    for rows, _, ssem in ring:
        _wait_scatter(rows, ssem)
    _drain_den()

    plsc.subcore_barrier()
    pltpu.sync_copy(num_sp.at[pl.ds(s * RPT, RPT)],
                    nump_hbm.at[pl.ds(c * NPAD + s * RPT, RPT)])
    pltpu.sync_copy(den_sp.at[pl.ds(s * RPT, RPT)],
                    denp_hbm.at[c, pl.ds(s * RPT, RPT)])


@functools.partial(
    pl.kernel,
    out_type=(jax.ShapeDtypeStruct((NW, NCH_S, BS), f32),
              jax.ShapeDtypeStruct((NW, NCH_S, BS), f32)),
    mesh=_sc_mesh,
    compiler_params=pltpu.CompilerParams(needs_layout_passes=False),
    scratch_types=[
        pltpu.VMEM((NPAD * 4,), f32),
        pltpu.VMEM((NCH_S, BS), i32),
        pltpu.VMEM((NCH_S, BS), i32),
        pltpu.VMEM((NCH_S, BS), f32),
        pltpu.VMEM((NCH_S, BS), f32),
    ],
)
def _sc_score(uv_hbm, src_hbm, dst_hbm, s0_hbm, s1_hbm,
              uv_tab, src_idx, dst_idx, s0buf, s1buf):
    c = lax.axis_index("c")
    s = lax.axis_index("s")
    g = c * NS + s
    pltpu.sync_copy(uv_hbm, uv_tab)
    pltpu.sync_copy(src_hbm.at[g], src_idx)
    pltpu.sync_copy(dst_hbm.at[g], dst_idx)

    @pl.loop(0, NCH_S)
    def _lp(j):
        for k in range(BS // 16):
            off = k * 16
            s16 = src_idx[j, pl.ds(off, 16)] * 4
            d16 = dst_idx[j, pl.ds(off, 16)] * 4
            u0 = plsc.load_gather(uv_tab, [s16])
            u1 = plsc.load_gather(uv_tab, [s16 + 1])
            v0 = plsc.load_gather(uv_tab, [d16 + 2])
            v1 = plsc.load_gather(uv_tab, [d16 + 3])
            s0buf[j, pl.ds(off, 16)] = u0 + v0
            s1buf[j, pl.ds(off, 16)] = u1 + v1

    pltpu.sync_copy(s0buf, s0_hbm.at[g])
    pltpu.sync_copy(s1buf, s1_hbm.at[g])


BM = 1024
NBLK = NPAD // BM


def _tc_dense1(x, W, attc):
    def body(x_ref, w_ref, att_ref, h_ref, a_ref):
        h = jnp.dot(x_ref[...], w_ref[...], preferred_element_type=f32)
        h_ref[...] = h
        a_ref[...] = jnp.dot(h, att_ref[...], preferred_element_type=f32)

    return pl.pallas_call(
        body,
        grid=(NBLK,),
        in_specs=[pl.BlockSpec((BM, 128), lambda i: (i, 0)),
                  pl.BlockSpec((128, 128), lambda i: (0, 0)),
                  pl.BlockSpec((128, 2), lambda i: (0, 0))],
        out_specs=[pl.BlockSpec((BM, 128), lambda i: (i, 0)),
                   pl.BlockSpec((BM, 2), lambda i: (i, 0))],
        out_shape=(jax.ShapeDtypeStruct((NPAD, 128), f32),
                   jax.ShapeDtypeStruct((NPAD, 2), f32)),
    )(x, W, attc)


def _combine(n0_ref, n1_ref, d0_ref, d1_ref, b_ref):
    num = n0_ref[...] + n1_ref[...]
    den = d0_ref[...] + d1_ref[...]
    z = num / (den + 1e-16) + b_ref[...]
    return jnp.where(z > 0, z, jnp.exp(z) - 1.0)


def _tc_combine_dense(nump, den0, den1, bias, W, attc):
    def body(n0_ref, n1_ref, d0_ref, d1_ref, b_ref, w_ref, att_ref,
             h_ref, a_ref):
        hin = _combine(n0_ref, n1_ref, d0_ref, d1_ref, b_ref)
        h = jnp.dot(hin, w_ref[...], preferred_element_type=f32)
        h_ref[...] = h
        a_ref[...] = jnp.dot(h, att_ref[...], preferred_element_type=f32)

    return pl.pallas_call(
        body,
        grid=(NBLK,),
        in_specs=[pl.BlockSpec((BM, 128), lambda i: (i, 0)),
                  pl.BlockSpec((BM, 128), lambda i: (i + NBLK, 0)),
                  pl.BlockSpec((BM, 1), lambda i: (i, 0)),
                  pl.BlockSpec((BM, 1), lambda i: (i, 0)),
                  pl.BlockSpec((1, 128), lambda i: (0, 0)),
                  pl.BlockSpec((128, 128), lambda i: (0, 0)),
                  pl.BlockSpec((128, 2), lambda i: (0, 0))],
        out_specs=[pl.BlockSpec((BM, 128), lambda i: (i, 0)),
                   pl.BlockSpec((BM, 2), lambda i: (i, 0))],
        out_shape=(jax.ShapeDtypeStruct((NPAD, 128), f32),
                   jax.ShapeDtypeStruct((NPAD, 2), f32)),
    )(nump, nump, den0, den1, bias, W, attc)


def _tc_combine_proj(nump, den0, den1, bias, Wcat, bcat):
    def body(n0_ref, n1_ref, d0_ref, d1_ref, b_ref, w_ref, bc_ref, o_ref):
        hf = _combine(n0_ref, n1_ref, d0_ref, d1_ref, b_ref)
        o_ref[...] = (jnp.dot(hf, w_ref[...], preferred_element_type=f32)
                      + bc_ref[...])

    return pl.pallas_call(
        body,
        grid=(NBLK,),
        in_specs=[pl.BlockSpec((BM, 128), lambda i: (i, 0)),
                  pl.BlockSpec((BM, 128), lambda i: (i + NBLK, 0)),
                  pl.BlockSpec((BM, 1), lambda i: (i, 0)),
                  pl.BlockSpec((BM, 1), lambda i: (i, 0)),
                  pl.BlockSpec((1, 128), lambda i: (0, 0)),
                  pl.BlockSpec((128, 4), lambda i: (0, 0)),
                  pl.BlockSpec((1, 4), lambda i: (0, 0))],
        out_specs=pl.BlockSpec((BM, 4), lambda i: (i, 0)),
        out_shape=jax.ShapeDtypeStruct((NPAD, 4), f32),
    )(nump, nump, den0, den1, bias, Wcat, bcat)


def _tc_logsm(s0, s1):
    rows = EPAD // BS

    def body(s0_ref, s1_ref, o0_ref, o1_ref):
        a = s0_ref[...]
        b = s1_ref[...]
        m = jnp.maximum(a, b)
        mn = jnp.minimum(a, b)
        l = m + jnp.log(1.0 + jnp.exp(mn - m))
        o0_ref[...] = a - l
        o1_ref[...] = b - l

    bm = 512
    return pl.pallas_call(
        body,
        grid=(rows // bm,),
        in_specs=[pl.BlockSpec((bm, BS), lambda i: (i, 0)),
                  pl.BlockSpec((bm, BS), lambda i: (i, 0))],
        out_specs=[pl.BlockSpec((bm, BS), lambda i: (i, 0)),
                   pl.BlockSpec((bm, BS), lambda i: (i, 0))],
        out_shape=(jax.ShapeDtypeStruct((rows, BS), f32),
                   jax.ShapeDtypeStruct((rows, BS), f32)),
    )(s0, s1)


def kernel(x, edge_index, W1, att_src1, att_dst1, b1,
           W2, att_src2, att_dst2, b2, W_fc, b_fc):
    x_pad = jnp.pad(x, ((0, NPAD - N), (0, 0)))
    npad_e = EPAD - E
    pad_dst = (N + (jnp.arange(npad_e, dtype=i32) % (NPAD - N))).astype(i32)
    src_all = jnp.concatenate([edge_index[0], pad_dst])
    dst_all = jnp.concatenate([edge_index[1], pad_dst])
    src_e = src_all.reshape(NW, NCH_E, BE)
    dst_e = dst_all.reshape(NW, NCH_E, BE)
    src_r = src_all.reshape(NW, NCH_S, BS)
    dst_r = dst_all.reshape(NW, NCH_S, BS)

    attc1 = jnp.stack([att_src1.reshape(-1), att_dst1.reshape(-1)], axis=1)
    attc2 = jnp.stack([att_src2.reshape(-1), att_dst2.reshape(-1)], axis=1)
    b1r = b1.reshape(1, 128)
    b2r = b2.reshape(1, 128)
    Wcat = jnp.concatenate([W_fc[:128], W_fc[128:]], axis=1)
    bcat = jnp.concatenate([b_fc, jnp.zeros((2,), f32)]).reshape(1, 4)

    h1, a1 = _tc_dense1(x_pad, W1, attc1)
    nump1, denp1 = _sc_edge(h1, a1.reshape(-1), src_e, dst_e)
    d10 = denp1[0].reshape(NPAD, 1)
    d11 = denp1[1].reshape(NPAD, 1)
    h2, a2 = _tc_combine_dense(nump1, d10, d11, b1r, W2, attc2)
    nump2, denp2 = _sc_edge(h2, a2.reshape(-1), src_e, dst_e)
    d20 = denp2[0].reshape(NPAD, 1)
    d21 = denp2[1].reshape(NPAD, 1)
    uv = _tc_combine_proj(nump2, d20, d21, b2r, Wcat, bcat)
    s0, s1 = _sc_score(uv.reshape(-1), src_r, dst_r)
    o0, o1 = _tc_logsm(s0.reshape(EPAD // BS, BS), s1.reshape(EPAD // BS, BS))
    return jnp.stack([o0.reshape(EPAD)[:E], o1.reshape(EPAD)[:E]], axis=1)


# speedup vs baseline: 42.4568x; 1.0128x over previous
import functools

import jax
import jax.numpy as jnp
from jax import lax
from jax.experimental import pallas as pl
from jax.experimental.pallas import tpu as pltpu
from jax.experimental.pallas import tpu_sc as plsc

N = 10000
E = 320000
NPAD = 10240
NS = 16
NW = 32
EPT = 10240
EPAD = NW * EPT
RPT = NPAD // NS

BE = 64
SB = 4
NCH_E = EPT // BE
NSB = NCH_E // SB

BS = 128
NCH_S = EPT // BS

f32 = jnp.float32
i32 = jnp.int32

_sc_mesh = plsc.VectorSubcoreMesh(core_axis_name="c", subcore_axis_name="s")


@functools.partial(
    pl.kernel,
    out_type=(jax.ShapeDtypeStruct((2 * NPAD, 128), f32),
              jax.ShapeDtypeStruct((2, NPAD), f32)),
    mesh=_sc_mesh,
    compiler_params=pltpu.CompilerParams(needs_layout_passes=False),
    scratch_types=[
        pltpu.VMEM((SB, BE), i32),
        pltpu.VMEM((SB, BE), i32),
        pltpu.VMEM((SB, BE), i32),
        pltpu.VMEM((SB, BE), i32),
        pltpu.VMEM((SB, BE), f32),
        pltpu.VMEM((BE, 128), f32),
        pltpu.VMEM((BE, 128), f32),
        pltpu.VMEM((BE, 128), f32),
        pltpu.VMEM((2 * NPAD,), f32),
        pltpu.VMEM_SHARED((NPAD, 128), f32),
        pltpu.VMEM_SHARED((NPAD,), f32),
        pltpu.SemaphoreType.DMA,
        pltpu.SemaphoreType.DMA,
        pltpu.SemaphoreType.DMA,
        pltpu.SemaphoreType.DMA,
        pltpu.SemaphoreType.DMA,
        pltpu.SemaphoreType.DMA,
        pltpu.SemaphoreType.DMA,
        pltpu.SemaphoreType.DMA,
        pltpu.SemaphoreType.DMA,
    ],
)
def _sc_edge(h_hbm, a_hbm, src_hbm, dst_hbm, nump_hbm, denp_hbm,
             sblkA, dblkA, sblkB, dblkB, wblk, rows0, rows1, rows2,
             a_tab, num_sp, den_sp, gs0, gs1, gs2, ss0, ss1, ss2, dsem,
             isA, isB):
    c = lax.axis_index("c")
    s = lax.axis_index("s")
    g = c * NS + s

    pltpu.async_copy(a_hbm, a_tab, gs2)

    zv = jnp.zeros((16,), f32)

    @pl.loop(0, BE)
    def _zero_rows(r):
        for k in range(8):
            rows0[r, pl.ds(k * 16, 16)] = zv

    for q in range(RPT // BE):
        pltpu.async_copy(rows0, num_sp.at[pl.ds(s * RPT + q * BE, BE)], gs0)
    for r in range(RPT // 128):
        pltpu.async_copy(rows0.at[r], den_sp.at[pl.ds(s * RPT + r * 128, 128)],
                         gs1)

    pltpu.async_copy(src_hbm.at[g, pl.ds(0, SB)], sblkA, isA)
    pltpu.async_copy(dst_hbm.at[g, pl.ds(0, SB)], dblkA, isA)

    for q in range(RPT // BE):
        pltpu.make_async_copy(rows0, num_sp.at[pl.ds(s * RPT + q * BE, BE)],
                              gs0).wait()
    for r in range(RPT // 128):
        pltpu.make_async_copy(rows0.at[r],
                              den_sp.at[pl.ds(s * RPT + r * 128, 128)],
                              gs1).wait()
    pltpu.make_async_copy(a_hbm, a_tab, gs2).wait()

    plsc.subcore_barrier()

    iota = lax.iota(i32, 16)

    def _weights(b, j, sblk, dblk):
        base = g * EPT + (b * SB + j) * BE
        for q in range(BE // 16):
            off = q * 16
            s16 = sblk[j, pl.ds(off, 16)]
            d16 = dblk[j, pl.ds(off, 16)]
            va = plsc.load_gather(a_tab, [s16 * 2])
            vb = plsc.load_gather(a_tab, [d16 * 2 + 1])
            e = va + vb
            e = jnp.where(e > 0, e, 0.2 * e)
            w = jnp.exp(e)
            w = jnp.where(base + off + iota < E, w, 0.0)
            wblk[j, pl.ds(off, 16)] = w

    ring = ((rows0, gs0, ss0), (rows1, gs1, ss1), (rows2, gs2, ss2))

    def _start(j, rows, sem, sblk):
        pltpu.async_copy(h_hbm.at[sblk.at[j]], rows, sem)

    def _wait_gather(j, rows, sem, sblk):
        pltpu.make_async_copy(h_hbm.at[sblk.at[j]], rows, sem).wait()

    def _wait_scatter(rows, sem):
        pltpu.make_async_copy(rows, num_sp.at[dblkA.at[0]], sem).wait()

    def _drain_den():
        for _ in range(SB):
            pltpu.make_async_copy(wblk.at[0], den_sp.at[dblkA.at[0]],
                                  dsem).wait()

    def _scale(j, rows):
        @pl.loop(0, BE // 16)
        def _sc(q):
            w16 = wblk[j, pl.ds(q * 16, 16)]
            for t in range(16):
                wv = jnp.full((16,), w16[t])
                r = q * 16 + t
                for k in range(8):
                    sl = pl.ds(k * 16, 16)
                    rows[r, sl] = rows[r, sl] * wv

    def _prefetch(b, sblk, dblk, sem):
        pltpu.async_copy(src_hbm.at[g, pl.ds(b * SB, SB)], sblk, sem)
        pltpu.async_copy(dst_hbm.at[g, pl.ds(b * SB, SB)], dblk, sem)

    def _wait_prefetch(b, sblk, dblk, sem):
        pltpu.make_async_copy(src_hbm.at[g, pl.ds(b * SB, SB)], sblk,
                              sem).wait()
        pltpu.make_async_copy(dst_hbm.at[g, pl.ds(b * SB, SB)], dblk,
                              sem).wait()

    def _block(b, first, sblk, dblk, isem, nsblk, ndblk, nisem):
        if not first:
            for rows, _, ssem in ring:
                _wait_scatter(rows, ssem)
            _drain_den()

        _wait_prefetch(b, sblk, dblk, isem)

        @pl.when(b + 1 < NSB)
        def _pf_next():
            _prefetch(b + 1, nsblk, ndblk, nisem)

        _start(0, rows0, gs0, sblk)
        _start(1, rows1, gs1, sblk)

        for j in range(SB):
            rows, gsem, ssem = ring[j % 3]
            _weights(b, j, sblk, dblk)
            _wait_gather(j, rows, gsem, sblk)
            _scale(j, rows)
            pltpu.async_copy(rows, num_sp.at[dblk.at[j]], ssem, add=True)
            pltpu.async_copy(wblk.at[j], den_sp.at[dblk.at[j]], dsem,
                             add=True)
            if j + 2 < SB:
                nrows, ngsem, nssem = ring[(j + 2) % 3]
                if j + 2 >= 3:
                    _wait_scatter(nrows, nssem)
                _start(j + 2, nrows, ngsem, sblk)

    _block(0, True, sblkA, dblkA, isA, sblkB, dblkB, isB)

    @pl.loop(0, NSB // 2 - 1)
    def _pair(p):
        b1 = 2 * p + 1
        _block(b1, False, sblkB, dblkB, isB, sblkA, dblkA, isA)
        _block(b1 + 1, False, sblkA, dblkA, isA, sblkB, dblkB, isB)

    _block(NSB - 1, False, sblkB, dblkB, isB, sblkA, dblkA, isA)

    for rows, _, ssem in ring:
        _wait_scatter(rows, ssem)
    _drain_den()

    plsc.subcore_barrier()
    pltpu.sync_copy(num_sp.at[pl.ds(s * RPT, RPT)],
                    nump_hbm.at[pl.ds(c * NPAD + s * RPT, RPT)])
    pltpu.sync_copy(den_sp.at[pl.ds(s * RPT, RPT)],
                    denp_hbm.at[c, pl.ds(s * RPT, RPT)])


@functools.partial(
    pl.kernel,
    out_type=(jax.ShapeDtypeStruct((NW, NCH_S, BS), f32),
              jax.ShapeDtypeStruct((NW, NCH_S, BS), f32)),
    mesh=_sc_mesh,
    compiler_params=pltpu.CompilerParams(needs_layout_passes=False),
    scratch_types=[
        pltpu.VMEM((NPAD * 4,), f32),
        pltpu.VMEM((NCH_S, BS), i32),
        pltpu.VMEM((NCH_S, BS), i32),
        pltpu.VMEM((NCH_S, BS), f32),
        pltpu.VMEM((NCH_S, BS), f32),
    ],
)
def _sc_score(uv_hbm, src_hbm, dst_hbm, s0_hbm, s1_hbm,
              uv_tab, src_idx, dst_idx, s0buf, s1buf):
    c = lax.axis_index("c")
    s = lax.axis_index("s")
    g = c * NS + s
    pltpu.sync_copy(uv_hbm, uv_tab)
    pltpu.sync_copy(src_hbm.at[g], src_idx)
    pltpu.sync_copy(dst_hbm.at[g], dst_idx)

    @pl.loop(0, NCH_S)
    def _lp(j):
        for k in range(BS // 16):
            off = k * 16
            s16 = src_idx[j, pl.ds(off, 16)] * 4
            d16 = dst_idx[j, pl.ds(off, 16)] * 4
            u0 = plsc.load_gather(uv_tab, [s16])
            u1 = plsc.load_gather(uv_tab, [s16 + 1])
            v0 = plsc.load_gather(uv_tab, [d16 + 2])
            v1 = plsc.load_gather(uv_tab, [d16 + 3])
            s0buf[j, pl.ds(off, 16)] = u0 + v0
            s1buf[j, pl.ds(off, 16)] = u1 + v1

    pltpu.sync_copy(s0buf, s0_hbm.at[g])
    pltpu.sync_copy(s1buf, s1_hbm.at[g])


BM = 1024
NBLK = NPAD // BM


def _tc_dense1(x, W, attc):
    def body(x_ref, w_ref, att_ref, h_ref, a_ref):
        h = jnp.dot(x_ref[...], w_ref[...], preferred_element_type=f32)
        h_ref[...] = h
        a_ref[...] = jnp.dot(h, att_ref[...], preferred_element_type=f32)

    return pl.pallas_call(
        body,
        grid=(NBLK,),
        in_specs=[pl.BlockSpec((BM, 128), lambda i: (i, 0)),
                  pl.BlockSpec((128, 128), lambda i: (0, 0)),
                  pl.BlockSpec((128, 2), lambda i: (0, 0))],
        out_specs=[pl.BlockSpec((BM, 128), lambda i: (i, 0)),
                   pl.BlockSpec((BM, 2), lambda i: (i, 0))],
        out_shape=(jax.ShapeDtypeStruct((NPAD, 128), f32),
                   jax.ShapeDtypeStruct((NPAD, 2), f32)),
    )(x, W, attc)


def _combine(n0_ref, n1_ref, d0_ref, d1_ref, b_ref):
    num = n0_ref[...] + n1_ref[...]
    den = d0_ref[...] + d1_ref[...]
    z = num / (den + 1e-16) + b_ref[...]
    return jnp.where(z > 0, z, jnp.exp(z) - 1.0)


def _tc_combine_dense(nump, den0, den1, bias, W, attc):
    def body(n0_ref, n1_ref, d0_ref, d1_ref, b_ref, w_ref, att_ref,
             h_ref, a_ref):
        hin = _combine(n0_ref, n1_ref, d0_ref, d1_ref, b_ref)
        h = jnp.dot(hin, w_ref[...], preferred_element_type=f32)
        h_ref[...] = h
        a_ref[...] = jnp.dot(h, att_ref[...], preferred_element_type=f32)

    return pl.pallas_call(
        body,
        grid=(NBLK,),
        in_specs=[pl.BlockSpec((BM, 128), lambda i: (i, 0)),
                  pl.BlockSpec((BM, 128), lambda i: (i + NBLK, 0)),
                  pl.BlockSpec((BM, 1), lambda i: (i, 0)),
                  pl.BlockSpec((BM, 1), lambda i: (i, 0)),
                  pl.BlockSpec((1, 128), lambda i: (0, 0)),
                  pl.BlockSpec((128, 128), lambda i: (0, 0)),
                  pl.BlockSpec((128, 2), lambda i: (0, 0))],
        out_specs=[pl.BlockSpec((BM, 128), lambda i: (i, 0)),
                   pl.BlockSpec((BM, 2), lambda i: (i, 0))],
        out_shape=(jax.ShapeDtypeStruct((NPAD, 128), f32),
                   jax.ShapeDtypeStruct((NPAD, 2), f32)),
    )(nump, nump, den0, den1, bias, W, attc)


def _tc_combine_proj(nump, den0, den1, bias, Wcat, bcat):
    def body(n0_ref, n1_ref, d0_ref, d1_ref, b_ref, w_ref, bc_ref, o_ref):
        hf = _combine(n0_ref, n1_ref, d0_ref, d1_ref, b_ref)
        o_ref[...] = (jnp.dot(hf, w_ref[...], preferred_element_type=f32)
                      + bc_ref[...])

    return pl.pallas_call(
        body,
        grid=(NBLK,),
        in_specs=[pl.BlockSpec((BM, 128), lambda i: (i, 0)),
                  pl.BlockSpec((BM, 128), lambda i: (i + NBLK, 0)),
                  pl.BlockSpec((BM, 1), lambda i: (i, 0)),
                  pl.BlockSpec((BM, 1), lambda i: (i, 0)),
                  pl.BlockSpec((1, 128), lambda i: (0, 0)),
                  pl.BlockSpec((128, 4), lambda i: (0, 0)),
                  pl.BlockSpec((1, 4), lambda i: (0, 0))],
        out_specs=pl.BlockSpec((BM, 4), lambda i: (i, 0)),
        out_shape=jax.ShapeDtypeStruct((NPAD, 4), f32),
    )(nump, nump, den0, den1, bias, Wcat, bcat)


def _tc_logsm(s0, s1):
    rows = EPAD // BS

    def body(s0_ref, s1_ref, o0_ref, o1_ref):
        a = s0_ref[...]
        b = s1_ref[...]
        m = jnp.maximum(a, b)
        mn = jnp.minimum(a, b)
        l = m + jnp.log(1.0 + jnp.exp(mn - m))
        o0_ref[...] = a - l
        o1_ref[...] = b - l

    bm = 512
    return pl.pallas_call(
        body,
        grid=(rows // bm,),
        in_specs=[pl.BlockSpec((bm, BS), lambda i: (i, 0)),
                  pl.BlockSpec((bm, BS), lambda i: (i, 0))],
        out_specs=[pl.BlockSpec((bm, BS), lambda i: (i, 0)),
                   pl.BlockSpec((bm, BS), lambda i: (i, 0))],
        out_shape=(jax.ShapeDtypeStruct((rows, BS), f32),
                   jax.ShapeDtypeStruct((rows, BS), f32)),
    )(s0, s1)


def kernel(x, edge_index, W1, att_src1, att_dst1, b1,
           W2, att_src2, att_dst2, b2, W_fc, b_fc):
    x_pad = jnp.pad(x, ((0, NPAD - N), (0, 0)))
    npad_e = EPAD - E
    pad_dst = (N + (jnp.arange(npad_e, dtype=i32) % (NPAD - N))).astype(i32)
    src_all = jnp.concatenate([edge_index[0], pad_dst])
    dst_all = jnp.concatenate([edge_index[1], pad_dst])
    src_e = src_all.reshape(NW, NCH_E, BE)
    dst_e = dst_all.reshape(NW, NCH_E, BE)
    src_r = src_all.reshape(NW, NCH_S, BS)
    dst_r = dst_all.reshape(NW, NCH_S, BS)

    attc1 = jnp.stack([att_src1.reshape(-1), att_dst1.reshape(-1)], axis=1)
    attc2 = jnp.stack([att_src2.reshape(-1), att_dst2.reshape(-1)], axis=1)
    b1r = b1.reshape(1, 128)
    b2r = b2.reshape(1, 128)
    Wcat = jnp.concatenate([W_fc[:128], W_fc[128:]], axis=1)
    bcat = jnp.concatenate([b_fc, jnp.zeros((2,), f32)]).reshape(1, 4)

    h1, a1 = _tc_dense1(x_pad, W1, attc1)
    nump1, denp1 = _sc_edge(h1, a1.reshape(-1), src_e, dst_e)
    d10 = denp1[0].reshape(NPAD, 1)
    d11 = denp1[1].reshape(NPAD, 1)
    h2, a2 = _tc_combine_dense(nump1, d10, d11, b1r, W2, attc2)
    nump2, denp2 = _sc_edge(h2, a2.reshape(-1), src_e, dst_e)
    d20 = denp2[0].reshape(NPAD, 1)
    d21 = denp2[1].reshape(NPAD, 1)
    uv = _tc_combine_proj(nump2, d20, d21, b2r, Wcat, bcat)
    s0, s1 = _sc_score(uv.reshape(-1), src_r, dst_r)
    o0, o1 = _tc_logsm(s0.reshape(EPAD // BS, BS), s1.reshape(EPAD // BS, BS))
    return jnp.stack([o0.reshape(EPAD)[:E], o1.reshape(EPAD)[:E]], axis=1)


# speedup vs baseline: 42.5048x; 1.0011x over previous
import functools

import jax
import jax.numpy as jnp
from jax import lax
from jax.experimental import pallas as pl
from jax.experimental.pallas import tpu as pltpu
from jax.experimental.pallas import tpu_sc as plsc

N = 10000
E = 320000
NPAD = 10240
NS = 16
NW = 32
EPT = 10240
EPAD = NW * EPT
RPT = NPAD // NS

BE = 64
SB = 4
NCH_E = EPT // BE
NSB = NCH_E // SB

BS = 128
NCH_S = EPT // BS

f32 = jnp.float32
i32 = jnp.int32

_sc_mesh = plsc.VectorSubcoreMesh(core_axis_name="c", subcore_axis_name="s")


@functools.partial(
    pl.kernel,
    out_type=(jax.ShapeDtypeStruct((2 * NPAD, 128), f32),
              jax.ShapeDtypeStruct((2, NPAD), f32)),
    mesh=_sc_mesh,
    compiler_params=pltpu.CompilerParams(needs_layout_passes=False),
    scratch_types=[
        pltpu.VMEM((SB, BE), i32),
        pltpu.VMEM((SB, BE), i32),
        pltpu.VMEM((SB, BE), i32),
        pltpu.VMEM((SB, BE), i32),
        pltpu.VMEM((SB, BE), f32),
        pltpu.VMEM((BE, 128), f32),
        pltpu.VMEM((BE, 128), f32),
        pltpu.VMEM((BE, 128), f32),
        pltpu.VMEM((2 * NPAD,), f32),
        pltpu.VMEM_SHARED((NPAD, 128), f32),
        pltpu.VMEM_SHARED((NPAD,), f32),
        pltpu.SemaphoreType.DMA,
        pltpu.SemaphoreType.DMA,
        pltpu.SemaphoreType.DMA,
        pltpu.SemaphoreType.DMA,
        pltpu.SemaphoreType.DMA,
        pltpu.SemaphoreType.DMA,
        pltpu.SemaphoreType.DMA,
        pltpu.SemaphoreType.DMA,
        pltpu.SemaphoreType.DMA,
    ],
)
def _sc_edge(h_hbm, a_hbm, src_hbm, dst_hbm, nump_hbm, denp_hbm,
             sblkA, dblkA, sblkB, dblkB, wblk, rows0, rows1, rows2,
             a_tab, num_sp, den_sp, gs0, gs1, gs2, ss0, ss1, ss2, dsem,
             isA, isB):
    c = lax.axis_index("c")
    s = lax.axis_index("s")
    g = c * NS + s

    pltpu.async_copy(a_hbm, a_tab, gs2)

    zv = jnp.zeros((16,), f32)

    @pl.loop(0, BE)
    def _zero_rows(r):
        for k in range(8):
            rows0[r, pl.ds(k * 16, 16)] = zv

    for q in range(RPT // BE):
        pltpu.async_copy(rows0, num_sp.at[pl.ds(s * RPT + q * BE, BE)], gs0)
    for r in range(RPT // 128):
        pltpu.async_copy(rows0.at[r], den_sp.at[pl.ds(s * RPT + r * 128, 128)],
                         gs1)

    pltpu.async_copy(src_hbm.at[g, pl.ds(0, SB)], sblkA, isA)
    pltpu.async_copy(dst_hbm.at[g, pl.ds(0, SB)], dblkA, isA)

    for q in range(RPT // BE):
        pltpu.make_async_copy(rows0, num_sp.at[pl.ds(s * RPT + q * BE, BE)],
                              gs0).wait()
    for r in range(RPT // 128):
        pltpu.make_async_copy(rows0.at[r],
                              den_sp.at[pl.ds(s * RPT + r * 128, 128)],
                              gs1).wait()
    pltpu.make_async_copy(a_hbm, a_tab, gs2).wait()

    plsc.subcore_barrier()

    iota = lax.iota(i32, 16)

    def _weights(b, j, sblk, dblk):
        base = g * EPT + (b * SB + j) * BE
        for q in range(BE // 16):
            off = q * 16
            s16 = sblk[j, pl.ds(off, 16)]
            d16 = dblk[j, pl.ds(off, 16)]
            va = plsc.load_gather(a_tab, [s16 * 2])
            vb = plsc.load_gather(a_tab, [d16 * 2 + 1])
            e = va + vb
            e = jnp.where(e > 0, e, 0.2 * e)
            w = jnp.exp(e)
            w = jnp.where(base + off + iota < E, w, 0.0)
            wblk[j, pl.ds(off, 16)] = w

    ring = ((rows0, gs0, ss0), (rows1, gs1, ss1), (rows2, gs2, ss2))

    def _start(j, rows, sem, sblk):
        pltpu.async_copy(h_hbm.at[sblk.at[j]], rows, sem)

    def _wait_gather(j, rows, sem, sblk):
        pltpu.make_async_copy(h_hbm.at[sblk.at[j]], rows, sem).wait()

    def _wait_scatter(rows, sem):
        pltpu.make_async_copy(rows, num_sp.at[dblkA.at[0]], sem).wait()

    def _drain_den():
        for _ in range(SB):
            pltpu.make_async_copy(wblk.at[0], den_sp.at[dblkA.at[0]],
                                  dsem).wait()

    def _scale(j, rows):
        @pl.loop(0, BE // 16)
        def _sc(q):
            w16 = wblk[j, pl.ds(q * 16, 16)]
            for t in range(16):
                wv = jnp.full((16,), w16[t])
                r = q * 16 + t
                for k in range(8):
                    sl = pl.ds(k * 16, 16)
                    rows[r, sl] = rows[r, sl] * wv

    def _prefetch(b, sblk, dblk, sem):
        pltpu.async_copy(src_hbm.at[g, pl.ds(b * SB, SB)], sblk, sem)
        pltpu.async_copy(dst_hbm.at[g, pl.ds(b * SB, SB)], dblk, sem)

    def _wait_prefetch(b, sblk, dblk, sem):
        pltpu.make_async_copy(src_hbm.at[g, pl.ds(b * SB, SB)], sblk,
                              sem).wait()
        pltpu.make_async_copy(dst_hbm.at[g, pl.ds(b * SB, SB)], dblk,
                              sem).wait()

    def _block(b, first, sblk, dblk, isem, nsblk, ndblk, nisem):
        if not first:
            for rows, _, ssem in ring:
                _wait_scatter(rows, ssem)
            _drain_den()

        _wait_prefetch(b, sblk, dblk, isem)

        @pl.when(b + 1 < NSB)
        def _pf_next():
            _prefetch(b + 1, nsblk, ndblk, nisem)

        _start(0, rows0, gs0, sblk)
        _start(1, rows1, gs1, sblk)

        for j in range(SB):
            rows, gsem, ssem = ring[j % 3]
            _weights(b, j, sblk, dblk)
            _wait_gather(j, rows, gsem, sblk)
            _scale(j, rows)
            pltpu.async_copy(rows, num_sp.at[dblk.at[j]], ssem, add=True)
            pltpu.async_copy(wblk.at[j], den_sp.at[dblk.at[j]], dsem,
                             add=True)
            if j + 2 < SB:
                nrows, ngsem, nssem = ring[(j + 2) % 3]
                if j + 2 >= 3:
                    _wait_scatter(nrows, nssem)
                _start(j + 2, nrows, ngsem, sblk)

    _block(0, True, sblkA, dblkA, isA, sblkB, dblkB, isB)

    @pl.loop(0, NSB // 2 - 1)
    def _pair(p):
        b1 = 2 * p + 1
        _block(b1, False, sblkB, dblkB, isB, sblkA, dblkA, isA)
        _block(b1 + 1, False, sblkA, dblkA, isA, sblkB, dblkB, isB)

    _block(NSB - 1, False, sblkB, dblkB, isB, sblkA, dblkA, isA)

    for rows, _, ssem in ring:
        _wait_scatter(rows, ssem)
    _drain_den()

    plsc.subcore_barrier()
    pltpu.sync_copy(num_sp.at[pl.ds(s * RPT, RPT)],
                    nump_hbm.at[pl.ds(c * NPAD + s * RPT, RPT)])
    pltpu.sync_copy(den_sp.at[pl.ds(s * RPT, RPT)],
                    denp_hbm.at[c, pl.ds(s * RPT, RPT)])


@functools.partial(
    pl.kernel,
    out_type=(jax.ShapeDtypeStruct((NW, NCH_S, BS), f32),
              jax.ShapeDtypeStruct((NW, NCH_S, BS), f32)),
    mesh=_sc_mesh,
    compiler_params=pltpu.CompilerParams(needs_layout_passes=False),
    scratch_types=[
        pltpu.VMEM((NPAD * 4,), f32),
        pltpu.VMEM((NCH_S, BS), i32),
        pltpu.VMEM((NCH_S, BS), i32),
        pltpu.VMEM((NCH_S, BS), f32),
        pltpu.VMEM((NCH_S, BS), f32),
    ],
)
def _sc_score(uv_hbm, src_hbm, dst_hbm, s0_hbm, s1_hbm,
              uv_tab, src_idx, dst_idx, s0buf, s1buf):
    c = lax.axis_index("c")
    s = lax.axis_index("s")
    g = c * NS + s
    pltpu.sync_copy(uv_hbm, uv_tab)
    pltpu.sync_copy(src_hbm.at[g], src_idx)
    pltpu.sync_copy(dst_hbm.at[g], dst_idx)

    @pl.loop(0, NCH_S)
    def _lp(j):
        for k in range(BS // 16):
            off = k * 16
            s16 = src_idx[j, pl.ds(off, 16)] * 4
            d16 = dst_idx[j, pl.ds(off, 16)] * 4
            u0 = plsc.load_gather(uv_tab, [s16])
            u1 = plsc.load_gather(uv_tab, [s16 + 1])
            v0 = plsc.load_gather(uv_tab, [d16 + 2])
            v1 = plsc.load_gather(uv_tab, [d16 + 3])
            s0buf[j, pl.ds(off, 16)] = u0 + v0
            s1buf[j, pl.ds(off, 16)] = u1 + v1

    pltpu.sync_copy(s0buf, s0_hbm.at[g])
    pltpu.sync_copy(s1buf, s1_hbm.at[g])


BM = 1024
NBLK = NPAD // BM


def _tc_dense1(x, W, attc):
    def body(x_ref, w_ref, att_ref, h_ref, a_ref):
        h = jnp.dot(x_ref[...], w_ref[...], preferred_element_type=f32)
        h_ref[...] = h
        a_ref[...] = jnp.dot(h, att_ref[...], preferred_element_type=f32)

    return pl.pallas_call(
        body,
        grid=(NBLK,),
        in_specs=[pl.BlockSpec((BM, 128), lambda i: (i, 0)),
                  pl.BlockSpec((128, 128), lambda i: (0, 0)),
                  pl.BlockSpec((128, 2), lambda i: (0, 0))],
        out_specs=[pl.BlockSpec((BM, 128), lambda i: (i, 0)),
                   pl.BlockSpec((BM, 2), lambda i: (i, 0))],
        out_shape=(jax.ShapeDtypeStruct((NPAD, 128), f32),
                   jax.ShapeDtypeStruct((NPAD, 2), f32)),
    )(x, W, attc)


def _combine(n0_ref, n1_ref, d0_ref, d1_ref, b_ref):
    num = n0_ref[...] + n1_ref[...]
    den = d0_ref[...] + d1_ref[...]
    z = num / (den + 1e-16) + b_ref[...]
    return jnp.where(z > 0, z, jnp.exp(z) - 1.0)


def _tc_combine_dense(nump, den0, den1, bias, W, attc):
    def body(n0_ref, n1_ref, d0_ref, d1_ref, b_ref, w_ref, att_ref,
             h_ref, a_ref):
        hin = _combine(n0_ref, n1_ref, d0_ref, d1_ref, b_ref)
        h = jnp.dot(hin, w_ref[...], preferred_element_type=f32)
        h_ref[...] = h
        a_ref[...] = jnp.dot(h, att_ref[...], preferred_element_type=f32)

    return pl.pallas_call(
        body,
        grid=(NBLK,),
        in_specs=[pl.BlockSpec((BM, 128), lambda i: (i, 0)),
                  pl.BlockSpec((BM, 128), lambda i: (i + NBLK, 0)),
                  pl.BlockSpec((BM, 1), lambda i: (i, 0)),
                  pl.BlockSpec((BM, 1), lambda i: (i, 0)),
                  pl.BlockSpec((1, 128), lambda i: (0, 0)),
                  pl.BlockSpec((128, 128), lambda i: (0, 0)),
                  pl.BlockSpec((128, 2), lambda i: (0, 0))],
        out_specs=[pl.BlockSpec((BM, 128), lambda i: (i, 0)),
                   pl.BlockSpec((BM, 2), lambda i: (i, 0))],
        out_shape=(jax.ShapeDtypeStruct((NPAD, 128), f32),
                   jax.ShapeDtypeStruct((NPAD, 2), f32)),
    )(nump, nump, den0, den1, bias, W, attc)


def _tc_combine_proj(nump, den0, den1, bias, Wcat, bcat):
    def body(n0_ref, n1_ref, d0_ref, d1_ref, b_ref, w_ref, bc_ref, o_ref):
        hf = _combine(n0_ref, n1_ref, d0_ref, d1_ref, b_ref)
        o_ref[...] = (jnp.dot(hf, w_ref[...], preferred_element_type=f32)
                      + bc_ref[...])

    return pl.pallas_call(
        body,
        grid=(NBLK,),
        in_specs=[pl.BlockSpec((BM, 128), lambda i: (i, 0)),
                  pl.BlockSpec((BM, 128), lambda i: (i + NBLK, 0)),
                  pl.BlockSpec((BM, 1), lambda i: (i, 0)),
                  pl.BlockSpec((BM, 1), lambda i: (i, 0)),
                  pl.BlockSpec((1, 128), lambda i: (0, 0)),
                  pl.BlockSpec((128, 4), lambda i: (0, 0)),
                  pl.BlockSpec((1, 4), lambda i: (0, 0))],
        out_specs=pl.BlockSpec((BM, 4), lambda i: (i, 0)),
        out_shape=jax.ShapeDtypeStruct((NPAD, 4), f32),
    )(nump, nump, den0, den1, bias, Wcat, bcat)


def _tc_logsm(s0, s1):
    rows = EPAD // BS

    def body(s0_ref, s1_ref, o0_ref, o1_ref):
        a = s0_ref[...]
        b = s1_ref[...]
        m = jnp.maximum(a, b)
        mn = jnp.minimum(a, b)
        l = m + jnp.log(1.0 + jnp.exp(mn - m))
        o0_ref[...] = a - l
        o1_ref[...] = b - l

    bm = 512
    return pl.pallas_call(
        body,
        grid=(rows // bm,),
        in_specs=[pl.BlockSpec((bm, BS), lambda i: (i, 0)),
                  pl.BlockSpec((bm, BS), lambda i: (i, 0))],
        out_specs=[pl.BlockSpec((bm, BS), lambda i: (i, 0)),
                   pl.BlockSpec((bm, BS), lambda i: (i, 0))],
        out_shape=(jax.ShapeDtypeStruct((rows, BS), f32),
                   jax.ShapeDtypeStruct((rows, BS), f32)),
    )(s0, s1)


def kernel(x, edge_index, W1, att_src1, att_dst1, b1,
           W2, att_src2, att_dst2, b2, W_fc, b_fc):
    x_pad = jnp.pad(x, ((0, NPAD - N), (0, 0)))
    npad_e = EPAD - E
    pad_dst = (N + (jnp.arange(npad_e, dtype=i32) % (NPAD - N))).astype(i32)
    src_all = jnp.concatenate([edge_index[0], pad_dst])
    dst_all = jnp.concatenate([edge_index[1], pad_dst])
    src_e = src_all.reshape(NW, NCH_E, BE)
    dst_e = dst_all.reshape(NW, NCH_E, BE)
    src_r = src_all.reshape(NW, NCH_S, BS)
    dst_r = dst_all.reshape(NW, NCH_S, BS)

    attc1 = jnp.stack([att_src1.reshape(-1), att_dst1.reshape(-1)], axis=1)
    attc2 = jnp.stack([att_src2.reshape(-1), att_dst2.reshape(-1)], axis=1)
    Wcat = jnp.concatenate([W_fc[:128], W_fc[128:]], axis=1)
    bcat = jnp.concatenate([b_fc, jnp.zeros((2,), f32)]).reshape(1, 4)

    b1r = b1.reshape(1, 128)
    b2r = b2.reshape(1, 128)

    h1, a1 = _tc_dense1(x_pad, W1, attc1)
    nump1, denp1 = _sc_edge(h1, a1.reshape(-1), src_e, dst_e)
    d10 = denp1[0].reshape(NPAD, 1)
    d11 = denp1[1].reshape(NPAD, 1)
    h2, a2 = _tc_combine_dense(nump1, d10, d11, b1r, W2, attc2)
    nump2, denp2 = _sc_edge(h2, a2.reshape(-1), src_e, dst_e)
    d20 = denp2[0].reshape(NPAD, 1)
    d21 = denp2[1].reshape(NPAD, 1)
    uv = _tc_combine_proj(nump2, d20, d21, b2r, Wcat, bcat)
    s0, s1 = _sc_score(uv.reshape(-1), src_r, dst_r)
    o0, o1 = _tc_logsm(s0.reshape(EPAD // BS, BS), s1.reshape(EPAD // BS, BS))
    return jnp.stack([o0.reshape(EPAD)[:E], o1.reshape(EPAD)[:E]], axis=1)
